# baseline (speedup 1.0000x reference)
import jax
import jax.numpy as jnp
from jax import lax
from jax.experimental import pallas as pl
from jax.experimental.pallas import tpu as pltpu
from jax.experimental.pallas import tpu_sc as plsc

B, L, H = 4096, 200, 64
BL = B * L
NC, NS = 2, 16
NW = NC * NS
PER_W = BL // NW
CHUNK = 128
NCHUNK = PER_W // CHUNK
K = NCHUNK // 2


NBUF = 8


def _body(x_hbm, tab_hbm, pos_hbm, out_hbm, idx_all, pos_rep, *bufsem):
    bufs = bufsem[:NBUF]
    sg = bufsem[NBUF:2 * NBUF]
    sw = bufsem[2 * NBUF:]
    wid = lax.axis_index("s") * NC + lax.axis_index("c")
    base = wid * PER_W

    pltpu.sync_copy(x_hbm.at[pl.ds(base, PER_W)], idx_all.at[0])
    pltpu.sync_copy(pos_hbm, pos_rep.at[pl.ds(0, L)])
    pltpu.sync_copy(pos_hbm, pos_rep.at[pl.ds(L, L)])

    def gather(c, buf, sem):
        pltpu.async_copy(tab_hbm.at[idx_all.at[0, pl.ds(c * CHUNK, CHUNK)]],
                         buf, sem)

    def gather_wait(c, buf, sem):
        pltpu.make_async_copy(
            tab_hbm.at[idx_all.at[0, pl.ds(c * CHUNK, CHUNK)]],
            buf, sem).wait()

    def write(c, buf, sem):
        pltpu.async_copy(buf, out_hbm.at[pl.ds(base + c * CHUNK, CHUNK)], sem)

    def write_wait(c, buf, sem):
        pltpu.make_async_copy(
            buf, out_hbm.at[pl.ds(base + c * CHUNK, CHUNK)], sem).wait()

    def round_body(k, carry):
        c0 = NBUF * k

        @pl.when(k >= 1)
        def _():
            for j in range(NBUF):
                write_wait(c0 - NBUF + j, bufs[j], sw[j])
        for j in range(NBUF):
            gather(c0 + j, bufs[j], sg[j])
        for j in range(NBUF):
            gather_wait(c0 + j, bufs[j], sg[j])
            l0 = lax.rem((c0 + j) * CHUNK, L)

            def row_add(i, carry2, buf=bufs[j], l0=l0):
                pr = l0 + i
                for q in range(H // 16):
                    plsc.addupdate(buf.at[i, pl.ds(16 * q, 16)],
                                   pos_rep[pr, pl.ds(16 * q, 16)])
                return carry2

            lax.fori_loop(0, CHUNK, row_add, 0)
            write(c0 + j, bufs[j], sw[j])
        return carry

    lax.fori_loop(0, NCHUNK // NBUF, round_body, 0)
    for j in range(NBUF):
        write_wait(NCHUNK - NBUF + j, bufs[j], sw[j])


def kernel(x, embed_table, pos_table):
    xf = x.reshape(BL)
    mesh = plsc.VectorSubcoreMesh(core_axis_name="c", subcore_axis_name="s")
    tok = pl.kernel(
        _body,
        out_type=jax.ShapeDtypeStruct((BL, H), jnp.float32),
        mesh=mesh,
        compiler_params=pltpu.CompilerParams(use_tc_tiling_on_sc=False,
                                             needs_layout_passes=False),
        scratch_types=(
            [pltpu.VMEM((1, PER_W), jnp.int32),
             pltpu.VMEM((2 * L, H), jnp.float32)]
            + [pltpu.VMEM((CHUNK, H), jnp.float32)] * NBUF
            + [pltpu.SemaphoreType.DMA] * (2 * NBUF)
        ),
    )(xf, embed_table, pos_table)
    return tok.reshape(B, L, H)


# speedup vs baseline: 1.1820x; 1.1820x over previous
import jax
import jax.numpy as jnp
from jax import lax
from jax.experimental import pallas as pl
from jax.experimental.pallas import tpu as pltpu
from jax.experimental.pallas import tpu_sc as plsc

B, L, H = 4096, 200, 64
BL = B * L
NC, NS = 2, 16
NW = NC * NS
PER_W = BL // NW
CHUNK = 128
NCHUNK = PER_W // CHUNK
K = NCHUNK // 2


NBUF = 8


def _body(x_hbm, tab_hbm, out_hbm, idx_all, *bufsem):
    bufs = bufsem[:NBUF]
    sg = bufsem[NBUF:2 * NBUF]
    sw = bufsem[2 * NBUF:]
    wid = lax.axis_index("s") * NC + lax.axis_index("c")
    base = wid * PER_W

    pltpu.sync_copy(x_hbm.at[pl.ds(base, PER_W)], idx_all.at[0])

    def gather(c, buf, sem):
        pltpu.async_copy(tab_hbm.at[idx_all.at[0, pl.ds(c * CHUNK, CHUNK)]],
                         buf, sem)

    def gather_wait(c, buf, sem):
        pltpu.make_async_copy(
            tab_hbm.at[idx_all.at[0, pl.ds(c * CHUNK, CHUNK)]],
            buf, sem).wait()

    def write(c, buf, sem):
        pltpu.async_copy(buf, out_hbm.at[pl.ds(base + c * CHUNK, CHUNK)], sem)

    def write_wait(c, buf, sem):
        pltpu.make_async_copy(
            buf, out_hbm.at[pl.ds(base + c * CHUNK, CHUNK)], sem).wait()

    def round_body(k, carry):
        c0 = NBUF * k

        @pl.when(k >= 1)
        def _():
            for j in range(NBUF):
                write_wait(c0 - NBUF + j, bufs[j], sw[j])
        for j in range(NBUF):
            gather(c0 + j, bufs[j], sg[j])
        for j in range(NBUF):
            gather_wait(c0 + j, bufs[j], sg[j])
            write(c0 + j, bufs[j], sw[j])
        return carry

    lax.fori_loop(0, NCHUNK // NBUF, round_body, 0)
    for j in range(NBUF):
        write_wait(NCHUNK - NBUF + j, bufs[j], sw[j])


def kernel(x, embed_table, pos_table):
    xf = x.reshape(BL)
    mesh = plsc.VectorSubcoreMesh(core_axis_name="c", subcore_axis_name="s")
    tok = pl.kernel(
        _body,
        out_type=jax.ShapeDtypeStruct((BL, H), jnp.float32),
        mesh=mesh,
        compiler_params=pltpu.CompilerParams(use_tc_tiling_on_sc=False,
                                             needs_layout_passes=False),
        scratch_types=(
            [pltpu.VMEM((1, PER_W), jnp.int32)]
            + [pltpu.VMEM((CHUNK, H), jnp.float32)] * NBUF
            + [pltpu.SemaphoreType.DMA] * (2 * NBUF)
        ),
    )(xf, embed_table)
    return tok.reshape(B, L, H) + pos_table[None, :, :]
